# Initial kernel scaffold; baseline (speedup 1.0000x reference)
import jax
import jax.numpy as jnp
from jax.experimental import pallas as pl


def kernel(x, edge_index, W_self0, W_neigh0, b0, W_self1, W_neigh1, b1, W_self2, W_neigh2, b2):
    raise NotImplementedError("write your pallas kernel here")


# speedup vs baseline: 5.0850x; 5.0850x over previous
import functools

import jax
import jax.numpy as jnp
from jax import lax
from jax.experimental import pallas as pl
from jax.experimental.pallas import tpu as pltpu
from jax.experimental.pallas import tpu_sc as plsc

_N = 10000
_E = 320000
_D = 128
_C = 47

_NC = 2
_NS = 16
_NW = _NC * _NS
_EPW = _E // _NW
_K = 80
_NCH = _EPW // _K
_NP = 10240
_RPT = _NP // _NS
_ZR = 32

_mesh = plsc.VectorSubcoreMesh(core_axis_name="c", subcore_axis_name="s")


def _zero_vmem_2d(ref, rows, cols):
    z = jnp.zeros((16,), jnp.float32)
    for r in range(rows):
        for c in range(cols // 16):
            ref[r, pl.ds(c * 16, 16)] = z


@functools.partial(
    pl.kernel,
    out_type=jax.ShapeDtypeStruct((_NC, _NP, _D), jnp.float32),
    mesh=_mesh,
    scratch_types=[
        pltpu.VMEM((_NCH, _K), jnp.int32),
        pltpu.VMEM((_NCH, _K), jnp.int32),
        pltpu.VMEM((_K, _D), jnp.float32),
        pltpu.VMEM((_ZR, _D), jnp.float32),
        pltpu.VMEM_SHARED((_NP, _D), jnp.float32),
        pltpu.SemaphoreType.DMA,
    ],
)
def _seg_sum(h_hbm, src_hbm, dst_hbm, out_hbm, srcv, dstv, rows, zbuf, acc, sem):
    cid = lax.axis_index("c")
    sid = lax.axis_index("s")
    wid = sid * _NC + cid

    _zero_vmem_2d(zbuf, _ZR, _D)

    def zero_body(j, _):
        pltpu.sync_copy(zbuf, acc.at[pl.ds(sid * _RPT + j * _ZR, _ZR)])
        return 0

    lax.fori_loop(0, _RPT // _ZR, zero_body, 0)

    pltpu.sync_copy(src_hbm.at[wid], srcv)
    pltpu.sync_copy(dst_hbm.at[wid], dstv)
    plsc.subcore_barrier()

    def edge_body(i, _):
        pltpu.async_copy(h_hbm.at[srcv.at[i]], rows, sem).wait()
        pltpu.sync_copy(rows, acc.at[dstv.at[i]], add=True)
        return 0

    lax.fori_loop(0, _NCH, edge_body, 0)
    plsc.subcore_barrier()

    def out_body(j, _):
        base = sid * _RPT + j * _ZR
        pltpu.sync_copy(acc.at[pl.ds(base, _ZR)], zbuf)
        pltpu.sync_copy(zbuf, out_hbm.at[cid, pl.ds(base, _ZR)])
        return 0

    lax.fori_loop(0, _RPT // _ZR, out_body, 0)


_BR = 400


def _dense_body(relu, h_ref, p0_ref, p1_ref, d0_ref, d1_ref, ws_ref, wn_ref,
                b_ref, o_ref):
    deg = d0_ref[...] + d1_ref[...]
    dinv = 1.0 / jnp.maximum(deg[:, 0:1], 1.0)
    agg = (p0_ref[...] + p1_ref[...]) * dinv
    o = (h_ref[...] @ ws_ref[...] + agg @ wn_ref[...] + b_ref[...])
    o_ref[...] = jnp.maximum(o, 0.0) if relu else o


def _dense(h, p0, p1, d0, d1, ws, wn, b, relu):
    grid = (_N // _BR,)
    return pl.pallas_call(
        functools.partial(_dense_body, relu),
        grid=grid,
        in_specs=[
            pl.BlockSpec((_BR, _D), lambda i: (i, 0)),
            pl.BlockSpec((_BR, _D), lambda i: (i, 0)),
            pl.BlockSpec((_BR, _D), lambda i: (i, 0)),
            pl.BlockSpec((_BR, 16), lambda i: (i, 0)),
            pl.BlockSpec((_BR, 16), lambda i: (i, 0)),
            pl.BlockSpec((_D, _D), lambda i: (0, 0)),
            pl.BlockSpec((_D, _D), lambda i: (0, 0)),
            pl.BlockSpec((1, _D), lambda i: (0, 0)),
        ],
        out_specs=pl.BlockSpec((_BR, _D), lambda i: (i, 0)),
        out_shape=jax.ShapeDtypeStruct((_N, _D), jnp.float32),
    )(h, p0, p1, d0, d1, ws, wn, b)


def _pad_cols(w, cols=_D):
    return jnp.pad(w, ((0, 0), (0, cols - w.shape[1])))


def kernel(x, edge_index, W_self0, W_neigh0, b0, W_self1, W_neigh1, b1,
           W_self2, W_neigh2, b2):
    src = edge_index[0].reshape(_NW, _NCH, _K)
    dst = edge_index[1].reshape(_NW, _NCH, _K)

    ones_tab = jnp.ones((_N, _D), jnp.float32)
    degp = _seg_sum(ones_tab, src, dst)
    d0, d1 = degp[0, :, :16], degp[1, :, :16]

    layers = [
        (W_self0, W_neigh0, b0.reshape(1, -1), True),
        (W_self1, W_neigh1, b1.reshape(1, -1), True),
        (_pad_cols(W_self2), _pad_cols(W_neigh2),
         _pad_cols(b2.reshape(1, -1)), False),
    ]

    h = x
    for ws, wn, b, relu in layers:
        parts = _seg_sum(h, src, dst)
        h = _dense(h, parts[0], parts[1], d0, d1, ws, wn, b, relu)
    return h[:, :_C]


# speedup vs baseline: 8.0944x; 1.5918x over previous
import functools

import jax
import jax.numpy as jnp
from jax import lax
from jax.experimental import pallas as pl
from jax.experimental.pallas import tpu as pltpu
from jax.experimental.pallas import tpu_sc as plsc

_N = 10000
_E = 320000
_D = 128
_C = 47

_NC = 2
_NS = 16
_NW = _NC * _NS
_EPW = _E // _NW
_K = 80
_NCH = _EPW // _K
_NP = 10240
_RPT = _NP // _NS
_ZR = 32

_mesh = plsc.VectorSubcoreMesh(core_axis_name="c", subcore_axis_name="s")


def _zero_vmem_2d(ref, rows, cols):
    z = jnp.zeros((16,), jnp.float32)
    for r in range(rows):
        for c in range(cols // 16):
            ref[r, pl.ds(c * 16, 16)] = z


@functools.partial(
    pl.kernel,
    out_type=jax.ShapeDtypeStruct((_NC, _NP, _D), jnp.float32),
    mesh=_mesh,
    scratch_types=[
        pltpu.VMEM((_EPW,), jnp.int32),
        pltpu.VMEM((_NCH, _K), jnp.int32),
        pltpu.VMEM((_K, _D), jnp.float32),
        pltpu.VMEM((_K, _D), jnp.float32),
        pltpu.VMEM_SHARED((_NP, _D), jnp.float32),
        pltpu.SemaphoreType.DMA,
        pltpu.SemaphoreType.DMA,
    ],
)
def _seg_sum(h_hbm, srcf_hbm, dst_hbm, out_hbm, srcv, dstv, rows_a, rows_b,
             acc, sem_a, sem_b):
    cid = lax.axis_index("c")
    sid = lax.axis_index("s")
    wid = sid * _NC + cid

    _zero_vmem_2d(rows_a, _K, _D)

    def zero_body(j, _):
        pltpu.sync_copy(rows_a, acc.at[pl.ds(sid * _RPT + j * _K, _K)])
        return 0

    lax.fori_loop(0, _RPT // _K, zero_body, 0)

    pltpu.sync_copy(srcf_hbm.at[pl.ds(wid * _EPW, _EPW)], srcv)
    pltpu.sync_copy(dst_hbm.at[wid], dstv)
    plsc.subcore_barrier()

    def gather(i, buf, sem):
        return pltpu.async_copy(h_hbm.at[srcv.at[pl.ds(i * _K, _K)]], buf, sem)

    def gwait(i, buf, sem):
        pltpu.make_async_copy(h_hbm.at[srcv.at[pl.ds(i * _K, _K)]], buf, sem).wait()

    gather(0, rows_a, sem_a)

    def edge_body(j, _):
        i0 = 2 * j
        gather(i0 + 1, rows_b, sem_b)
        gwait(i0, rows_a, sem_a)
        pltpu.sync_copy(rows_a, acc.at[dstv.at[i0]], add=True)
        gather(i0 + 2, rows_a, sem_a)
        gwait(i0 + 1, rows_b, sem_b)
        pltpu.sync_copy(rows_b, acc.at[dstv.at[i0 + 1]], add=True)
        return 0

    lax.fori_loop(0, _NCH // 2, edge_body, 0)
    gwait(_NCH - 1, rows_a, sem_a)
    pltpu.sync_copy(rows_a, acc.at[dstv.at[_NCH - 1]], add=True)
    plsc.subcore_barrier()

    def out_body(j, _):
        base = sid * _RPT + j * _K
        pltpu.sync_copy(acc.at[pl.ds(base, _K)], rows_a)
        pltpu.sync_copy(rows_a, out_hbm.at[cid, pl.ds(base, _K)])
        return 0

    lax.fori_loop(0, _RPT // _K, out_body, 0)


_BR = 400


def _dense_body(relu, h_ref, p0_ref, p1_ref, d0_ref, d1_ref, ws_ref, wn_ref,
                b_ref, o_ref):
    deg = d0_ref[...] + d1_ref[...]
    dinv = 1.0 / jnp.maximum(deg[:, 0:1], 1.0)
    agg = (p0_ref[...] + p1_ref[...]) * dinv
    o = (h_ref[...] @ ws_ref[...] + agg @ wn_ref[...] + b_ref[...])
    o_ref[...] = jnp.maximum(o, 0.0) if relu else o


def _dense(h, p0, p1, d0, d1, ws, wn, b, relu):
    grid = (_N // _BR,)
    return pl.pallas_call(
        functools.partial(_dense_body, relu),
        grid=grid,
        in_specs=[
            pl.BlockSpec((_BR, _D), lambda i: (i, 0)),
            pl.BlockSpec((_BR, _D), lambda i: (i, 0)),
            pl.BlockSpec((_BR, _D), lambda i: (i, 0)),
            pl.BlockSpec((_BR, 16), lambda i: (i, 0)),
            pl.BlockSpec((_BR, 16), lambda i: (i, 0)),
            pl.BlockSpec((_D, _D), lambda i: (0, 0)),
            pl.BlockSpec((_D, _D), lambda i: (0, 0)),
            pl.BlockSpec((1, _D), lambda i: (0, 0)),
        ],
        out_specs=pl.BlockSpec((_BR, _D), lambda i: (i, 0)),
        out_shape=jax.ShapeDtypeStruct((_N, _D), jnp.float32),
    )(h, p0, p1, d0, d1, ws, wn, b)


def _pad_cols(w, cols=_D):
    return jnp.pad(w, ((0, 0), (0, cols - w.shape[1])))


def kernel(x, edge_index, W_self0, W_neigh0, b0, W_self1, W_neigh1, b1,
           W_self2, W_neigh2, b2):
    src = edge_index[0]
    dst = edge_index[1].reshape(_NW, _NCH, _K)

    ones_tab = jnp.ones((_N, _D), jnp.float32)
    degp = _seg_sum(ones_tab, src, dst)
    d0, d1 = degp[0, :, :16], degp[1, :, :16]

    layers = [
        (W_self0, W_neigh0, b0.reshape(1, -1), True),
        (W_self1, W_neigh1, b1.reshape(1, -1), True),
        (_pad_cols(W_self2), _pad_cols(W_neigh2),
         _pad_cols(b2.reshape(1, -1)), False),
    ]

    h = x
    for ws, wn, b, relu in layers:
        parts = _seg_sum(h, src, dst)
        h = _dense(h, parts[0], parts[1], d0, d1, ws, wn, b, relu)
    return h[:, :_C]


# speedup vs baseline: 9.6821x; 1.1961x over previous
import functools

import jax
import jax.numpy as jnp
from jax import lax
from jax.experimental import pallas as pl
from jax.experimental.pallas import tpu as pltpu
from jax.experimental.pallas import tpu_sc as plsc

_N = 10000
_E = 320000
_D = 128
_C = 47

_NC = 2
_NS = 16
_NW = _NC * _NS
_EPW = _E // _NW
_K = 80
_NCH = _EPW // _K
_NP = 10240
_RPT = _NP // _NS
_ZR = 32

_mesh = plsc.VectorSubcoreMesh(core_axis_name="c", subcore_axis_name="s")


def _zero_vmem_2d(ref, rows, cols):
    z = jnp.zeros((16,), jnp.float32)
    for r in range(rows):
        for c in range(cols // 16):
            ref[r, pl.ds(c * 16, 16)] = z


@functools.partial(
    pl.kernel,
    out_type=jax.ShapeDtypeStruct((_NC, _NP, _D), jnp.float32),
    mesh=_mesh,
    compiler_params=pltpu.CompilerParams(needs_layout_passes=False),
    scratch_types=[
        pltpu.VMEM((_EPW,), jnp.int32),
        pltpu.VMEM((_NCH, _K), jnp.int32),
        pltpu.VMEM((_K, _D), jnp.float32),
        pltpu.VMEM((_K, _D), jnp.float32),
        pltpu.VMEM_SHARED((_NP, _D), jnp.float32),
        pltpu.SemaphoreType.DMA,
        pltpu.SemaphoreType.DMA,
    ],
)
def _seg_sum(h_hbm, srcf_hbm, dst_hbm, out_hbm, srcv, dstv, rows_a, rows_b,
             acc, sem_a, sem_b):
    cid = lax.axis_index("c")
    sid = lax.axis_index("s")
    wid = sid * _NC + cid

    _zero_vmem_2d(rows_a, _K, _D)

    def zero_body(j, _):
        pltpu.sync_copy(rows_a, acc.at[pl.ds(sid * _RPT + j * _K, _K)])
        return 0

    lax.fori_loop(0, _RPT // _K, zero_body, 0)

    pltpu.sync_copy(srcf_hbm.at[pl.ds(wid * _EPW, _EPW)], srcv)
    pltpu.sync_copy(dst_hbm.at[wid], dstv)
    plsc.subcore_barrier()

    def gather(i, buf, sem):
        return pltpu.async_copy(h_hbm.at[srcv.at[pl.ds(i * _K, _K)]], buf, sem)

    def gwait(i, buf, sem):
        pltpu.make_async_copy(h_hbm.at[srcv.at[pl.ds(i * _K, _K)]], buf, sem).wait()

    gather(0, rows_a, sem_a)

    def edge_body(j, _):
        i0 = 2 * j
        gather(i0 + 1, rows_b, sem_b)
        gwait(i0, rows_a, sem_a)
        pltpu.sync_copy(rows_a, acc.at[dstv.at[i0]], add=True)
        gather(i0 + 2, rows_a, sem_a)
        gwait(i0 + 1, rows_b, sem_b)
        pltpu.sync_copy(rows_b, acc.at[dstv.at[i0 + 1]], add=True)
        return 0

    lax.fori_loop(0, _NCH // 2, edge_body, 0)
    gwait(_NCH - 1, rows_a, sem_a)
    pltpu.sync_copy(rows_a, acc.at[dstv.at[_NCH - 1]], add=True)
    plsc.subcore_barrier()

    def out_body(j, _):
        base = sid * _RPT + j * _K
        pltpu.sync_copy(acc.at[pl.ds(base, _K)], rows_a)
        pltpu.sync_copy(rows_a, out_hbm.at[cid, pl.ds(base, _K)])
        return 0

    lax.fori_loop(0, _RPT // _K, out_body, 0)


_HR = _NP // 128
_HT = 8


@functools.partial(
    pl.kernel,
    out_type=jax.ShapeDtypeStruct((_NC, _HR // _HT, _HT, 128), jnp.float32),
    mesh=_mesh,
    compiler_params=pltpu.CompilerParams(needs_layout_passes=False),
    scratch_types=[
        pltpu.VMEM((_EPW // 16, 16), jnp.int32),
        pltpu.VMEM((_HR, 128), jnp.float32),
        pltpu.VMEM((_NS, _HT, 128), jnp.float32),
        pltpu.VMEM_SHARED((_NS, _HR, 128), jnp.float32),
    ],
)
def _deg_hist(dstf_hbm, out_hbm, dstv, hist, comb, shr):
    cid = lax.axis_index("c")
    sid = lax.axis_index("s")
    wid = sid * _NC + cid

    pltpu.sync_copy(dstf_hbm.at[wid], dstv)
    z = jnp.zeros((16,), jnp.float32)

    def zero_body(i, _):
        for c in range(8):
            hist[i, pl.ds(c * 16, 16)] = z
        return 0

    lax.fori_loop(0, _HR, zero_body, 0)

    one = jnp.ones((16,), jnp.float32)

    def edge_body(i, _):
        idx = dstv[i, pl.ds(0, 16)]
        plsc.addupdate_scatter(hist, [idx >> 7, idx & 127], one)
        return 0

    lax.fori_loop(0, _EPW // 16, edge_body, 0)

    pltpu.sync_copy(hist, shr.at[sid])
    plsc.subcore_barrier()

    @pl.when(sid < _HR // _HT)
    def _():
        for t in range(_NS):
            pltpu.sync_copy(shr.at[t, pl.ds(sid * _HT, _HT)], comb.at[t])

        def red_body(i, _):
            for c in range(8):
                a0 = jnp.zeros((16,), jnp.float32)

                def row(t, a):
                    return a + comb[t, i, pl.ds(c * 16, 16)]

                hist[i, pl.ds(c * 16, 16)] = lax.fori_loop(0, _NS, row, a0)
            return 0

        lax.fori_loop(0, _HT, red_body, 0)
        pltpu.sync_copy(hist.at[pl.ds(0, _HT)], out_hbm.at[cid, sid])


_BR = 400


def _dense_body(relu, h_ref, p0_ref, p1_ref, d0_ref, d1_ref, ws_ref, wn_ref,
                b_ref, o_ref):
    deg = d0_ref[...] + d1_ref[...]
    dinv = 1.0 / jnp.maximum(deg[:, 0:1], 1.0)
    agg = (p0_ref[...] + p1_ref[...]) * dinv
    o = (h_ref[...] @ ws_ref[...] + agg @ wn_ref[...] + b_ref[...])
    o_ref[...] = jnp.maximum(o, 0.0) if relu else o


def _dense(h, p0, p1, d0, d1, ws, wn, b, relu):
    grid = (_N // _BR,)
    return pl.pallas_call(
        functools.partial(_dense_body, relu),
        grid=grid,
        in_specs=[
            pl.BlockSpec((_BR, _D), lambda i: (i, 0)),
            pl.BlockSpec((_BR, _D), lambda i: (i, 0)),
            pl.BlockSpec((_BR, _D), lambda i: (i, 0)),
            pl.BlockSpec((_BR, 16), lambda i: (i, 0)),
            pl.BlockSpec((_BR, 16), lambda i: (i, 0)),
            pl.BlockSpec((_D, _D), lambda i: (0, 0)),
            pl.BlockSpec((_D, _D), lambda i: (0, 0)),
            pl.BlockSpec((1, _D), lambda i: (0, 0)),
        ],
        out_specs=pl.BlockSpec((_BR, _D), lambda i: (i, 0)),
        out_shape=jax.ShapeDtypeStruct((_N, _D), jnp.float32),
    )(h, p0, p1, d0, d1, ws, wn, b)


def _pad_cols(w, cols=_D):
    return jnp.pad(w, ((0, 0), (0, cols - w.shape[1])))


def kernel(x, edge_index, W_self0, W_neigh0, b0, W_self1, W_neigh1, b1,
           W_self2, W_neigh2, b2):
    src = edge_index[0]
    dst = edge_index[1].reshape(_NW, _NCH, _K)

    degp = _deg_hist(edge_index[1].reshape(_NW, _EPW // 16, 16))
    d0 = jnp.broadcast_to(degp[0].reshape(_NP, 1), (_NP, 16))
    d1 = jnp.broadcast_to(degp[1].reshape(_NP, 1), (_NP, 16))

    layers = [
        (W_self0, W_neigh0, b0.reshape(1, -1), True),
        (W_self1, W_neigh1, b1.reshape(1, -1), True),
        (_pad_cols(W_self2), _pad_cols(W_neigh2),
         _pad_cols(b2.reshape(1, -1)), False),
    ]

    h = x
    for ws, wn, b, relu in layers:
        parts = _seg_sum(h, src, dst)
        h = _dense(h, parts[0], parts[1], d0, d1, ws, wn, b, relu)
    return h[:, :_C]


# speedup vs baseline: 10.3420x; 1.0682x over previous
import functools

import jax
import jax.numpy as jnp
from jax import lax
from jax.experimental import pallas as pl
from jax.experimental.pallas import tpu as pltpu
from jax.experimental.pallas import tpu_sc as plsc

_N = 10000
_E = 320000
_D = 128
_C = 47

_NC = 2
_NS = 16
_NW = _NC * _NS
_EPW = _E // _NW
_K = 80
_NCH = _EPW // _K
_NP = 10240
_RPT = _NP // _NS
_ZR = 32

_mesh = plsc.VectorSubcoreMesh(core_axis_name="c", subcore_axis_name="s")


def _zero_vmem_2d(ref, rows, cols):
    z = jnp.zeros((16,), jnp.float32)
    for r in range(rows):
        for c in range(cols // 16):
            ref[r, pl.ds(c * 16, 16)] = z


@functools.partial(
    pl.kernel,
    out_type=jax.ShapeDtypeStruct((_NC, _NP, _D), jnp.float32),
    mesh=_mesh,
    compiler_params=pltpu.CompilerParams(needs_layout_passes=False),
    scratch_types=[
        pltpu.VMEM((_EPW,), jnp.int32),
        pltpu.VMEM((_NCH, _K), jnp.int32),
        pltpu.VMEM((_K, _D), jnp.float32),
        pltpu.VMEM((_K, _D), jnp.float32),
        pltpu.VMEM_SHARED((_NP, _D), jnp.float32),
        pltpu.SemaphoreType.DMA,
        pltpu.SemaphoreType.DMA,
    ],
)
def _seg_sum(h_hbm, srcf_hbm, dst_hbm, out_hbm, srcv, dstv, rows_a, rows_b,
             acc, sem_a, sem_b):
    cid = lax.axis_index("c")
    sid = lax.axis_index("s")
    wid = sid * _NC + cid

    _zero_vmem_2d(rows_a, _K, _D)

    def zero_body(j, _):
        pltpu.sync_copy(rows_a, acc.at[pl.ds(sid * _RPT + j * _K, _K)])
        return 0

    lax.fori_loop(0, _RPT // _K, zero_body, 0)

    pltpu.sync_copy(srcf_hbm.at[pl.ds(wid * _EPW, _EPW)], srcv)
    pltpu.sync_copy(dst_hbm.at[wid], dstv)
    plsc.subcore_barrier()

    def gather(i, buf, sem):
        return pltpu.async_copy(h_hbm.at[srcv.at[pl.ds(i * _K, _K)]], buf, sem)

    def gwait(i, buf, sem):
        pltpu.make_async_copy(h_hbm.at[srcv.at[pl.ds(i * _K, _K)]], buf, sem).wait()

    gather(0, rows_a, sem_a)

    def edge_body(j, _):
        i0 = 2 * j
        gather(i0 + 1, rows_b, sem_b)
        gwait(i0, rows_a, sem_a)
        pltpu.sync_copy(rows_a, acc.at[dstv.at[i0]], add=True)
        gather(i0 + 2, rows_a, sem_a)
        gwait(i0 + 1, rows_b, sem_b)
        pltpu.sync_copy(rows_b, acc.at[dstv.at[i0 + 1]], add=True)
        return 0

    lax.fori_loop(0, _NCH // 2, edge_body, 0)
    gwait(_NCH - 1, rows_a, sem_a)
    pltpu.sync_copy(rows_a, acc.at[dstv.at[_NCH - 1]], add=True)
    plsc.subcore_barrier()

    def out_body(j, _):
        base = sid * _RPT + j * _K
        pltpu.sync_copy(acc.at[pl.ds(base, _K)], rows_a)
        pltpu.sync_copy(rows_a, out_hbm.at[cid, pl.ds(base, _K)])
        return 0

    lax.fori_loop(0, _RPT // _K, out_body, 0)


_HR = _NP // 128
_HT = 8


@functools.partial(
    pl.kernel,
    out_type=jax.ShapeDtypeStruct((_NC, _HR // _HT, _HT, 128), jnp.float32),
    mesh=_mesh,
    compiler_params=pltpu.CompilerParams(needs_layout_passes=False),
    scratch_types=[
        pltpu.VMEM((_EPW // 16, 16), jnp.int32),
        pltpu.VMEM((_HR, 128), jnp.float32),
        pltpu.VMEM((_NS, _HT, 128), jnp.float32),
        pltpu.VMEM_SHARED((_NS, _HR, 128), jnp.float32),
    ],
)
def _deg_hist(dstf_hbm, out_hbm, dstv, hist, comb, shr):
    cid = lax.axis_index("c")
    sid = lax.axis_index("s")
    wid = sid * _NC + cid

    pltpu.sync_copy(dstf_hbm.at[wid], dstv)
    z = jnp.zeros((16,), jnp.float32)

    def zero_body(i, _):
        for c in range(8):
            hist[i, pl.ds(c * 16, 16)] = z
        return 0

    lax.fori_loop(0, _HR, zero_body, 0)

    one = jnp.ones((16,), jnp.float32)

    def edge_body(i, _):
        idx = dstv[i, pl.ds(0, 16)]
        plsc.addupdate_scatter(hist, [idx >> 7, idx & 127], one)
        return 0

    lax.fori_loop(0, _EPW // 16, edge_body, 0)

    pltpu.sync_copy(hist, shr.at[sid])
    plsc.subcore_barrier()

    @pl.when(sid < _HR // _HT)
    def _():
        for t in range(_NS):
            pltpu.sync_copy(shr.at[t, pl.ds(sid * _HT, _HT)], comb.at[t])

        def red_body(i, _):
            for c in range(8):
                a0 = jnp.zeros((16,), jnp.float32)

                def row(t, a):
                    return a + comb[t, i, pl.ds(c * 16, 16)]

                hist[i, pl.ds(c * 16, 16)] = lax.fori_loop(0, _NS, row, a0)
            return 0

        lax.fori_loop(0, _HT, red_body, 0)
        pltpu.sync_copy(hist.at[pl.ds(0, _HT)], out_hbm.at[cid, sid])


_BR = 2000


def _dense_body(relu, h_ref, p0_ref, p1_ref, d0_ref, d1_ref, ws_ref, wn_ref,
                b_ref, o_ref):
    deg = d0_ref[...] + d1_ref[...]
    dinv = 1.0 / jnp.maximum(deg[:, 0:1], 1.0)
    agg = (p0_ref[...] + p1_ref[...]) * dinv
    o = (h_ref[...] @ ws_ref[...] + agg @ wn_ref[...] + b_ref[...])
    o_ref[...] = jnp.maximum(o, 0.0) if relu else o


def _dense(h, p0, p1, d0, d1, ws, wn, b, relu):
    grid = (_N // _BR,)
    return pl.pallas_call(
        functools.partial(_dense_body, relu),
        grid=grid,
        in_specs=[
            pl.BlockSpec((_BR, _D), lambda i: (i, 0)),
            pl.BlockSpec((_BR, _D), lambda i: (i, 0)),
            pl.BlockSpec((_BR, _D), lambda i: (i, 0)),
            pl.BlockSpec((_BR, 16), lambda i: (i, 0)),
            pl.BlockSpec((_BR, 16), lambda i: (i, 0)),
            pl.BlockSpec((_D, _D), lambda i: (0, 0)),
            pl.BlockSpec((_D, _D), lambda i: (0, 0)),
            pl.BlockSpec((1, _D), lambda i: (0, 0)),
        ],
        out_specs=pl.BlockSpec((_BR, _D), lambda i: (i, 0)),
        out_shape=jax.ShapeDtypeStruct((_N, _D), jnp.float32),
    )(h, p0, p1, d0, d1, ws, wn, b)


def _pad_cols(w, cols=_D):
    return jnp.pad(w, ((0, 0), (0, cols - w.shape[1])))


def kernel(x, edge_index, W_self0, W_neigh0, b0, W_self1, W_neigh1, b1,
           W_self2, W_neigh2, b2):
    src = edge_index[0]
    dst = edge_index[1].reshape(_NW, _NCH, _K)

    degp = _deg_hist(edge_index[1].reshape(_NW, _EPW // 16, 16))
    d0 = jnp.broadcast_to(degp[0].reshape(_NP, 1), (_NP, 16))
    d1 = jnp.broadcast_to(degp[1].reshape(_NP, 1), (_NP, 16))

    layers = [
        (W_self0, W_neigh0, b0.reshape(1, -1), True),
        (W_self1, W_neigh1, b1.reshape(1, -1), True),
        (_pad_cols(W_self2), _pad_cols(W_neigh2),
         _pad_cols(b2.reshape(1, -1)), False),
    ]

    h = x
    for ws, wn, b, relu in layers:
        parts = _seg_sum(h, src, dst)
        h = _dense(h, parts[0], parts[1], d0, d1, ws, wn, b, relu)
    return h[:, :_C]
